# baseline (speedup 1.0000x reference)
import functools

import jax
import jax.numpy as jnp
from jax import lax
from jax.experimental import pallas as pl
from jax.experimental.pallas import tpu as pltpu
from jax.experimental.pallas import tpu_sc as plsc

NUM_CORES = 2
NUM_SUBCORES = 16
NW = NUM_CORES * NUM_SUBCORES
LANES = 16
SUB = 8
LANE = 128
NBUF = 4


def kernel(x, token_table, pos_table):
    B, L = x.shape
    V, D = token_table.shape
    BW = B // NW
    LP = ((L + LANE - 1) // LANE) * LANE
    LT = LP // LANE
    BT = BW // SUB
    assert B % NW == 0 and BW == LANE and L % NBUF == 0 and D % SUB == 0
    BG = BW // LANES
    DT = D // SUB

    mesh = plsc.VectorSubcoreMesh(core_axis_name="c", subcore_axis_name="s")

    @functools.partial(
        pl.kernel,
        out_type=jax.ShapeDtypeStruct((L, DT, NW, SUB, LANE), jnp.float32),
        mesh=mesh,
        scratch_types=[
            pltpu.VMEM((BT, 1, SUB, LANE), jnp.int32),
            pltpu.VMEM((L, BW), jnp.int32),
            pltpu.VMEM((L, D), jnp.float32),
            [pltpu.VMEM((BW, D), jnp.float32) for _ in range(NBUF)],
            [pltpu.VMEM((DT, 1, SUB, LANE), jnp.float32) for _ in range(NBUF)],
            [pltpu.SemaphoreType.DMA for _ in range(NBUF)],
            [pltpu.SemaphoreType.DMA for _ in range(NBUF)],
        ],
        compiler_params=pltpu.CompilerParams(
            use_tc_tiling_on_sc=False, needs_layout_passes=False
        ),
    )
    def emb_kernel(xt_hbm, tok_hbm, pos_hbm, out_hbm,
                   idx_raw, idx_v, pos_v, inbuf, tbuf, gsem, wsem):
        wid = lax.axis_index("s") * NUM_CORES + lax.axis_index("c")
        iota = lax.iota(jnp.int32, LANES)

        pltpu.sync_copy(pos_hbm, pos_v)

        rt = [iota // SUB + jnp.int32(g * (LANES // SUB)) for g in range(BG)]
        rr = iota % SUB
        zt = jnp.zeros((LANES,), jnp.int32)
        for h in range(LT):
            pltpu.sync_copy(
                xt_hbm.at[pl.ds(wid * BT, BT), pl.ds(h, 1)], idx_raw
            )
            lhi = min(L, (h + 1) * LANE)

            @plsc.parallel_loop(h * LANE, lhi, unroll=2)
            def detile_body(l):
                cc = jnp.full((LANES,), l - h * LANE, jnp.int32)
                for g in range(BG):
                    idx_v[l, pl.ds(g * LANES, LANES)] = plsc.load_gather(
                        idx_raw, [rt[g], zt, rr, cc]
                    )

        def fire_gather(l, b):
            pltpu.async_copy(tok_hbm.at[idx_v.at[l]], inbuf[b], gsem[b])

        def wait_gather(b):
            pltpu.make_async_copy(tok_hbm.at[pl.ds(0, BW)], inbuf[b], gsem[b]).wait()

        def transpose_add(l, b):
            src = inbuf[b]
            dst = tbuf[b]
            rowbase = [iota + jnp.int32(g * LANES) for g in range(BG)]
            lrows = jnp.full((LANES,), l, jnp.int32)

            @plsc.parallel_loop(0, D, unroll=2)
            def d_body(d):
                cols = jnp.full((LANES,), d, jnp.int32)
                pv = plsc.load_gather(pos_v, [lrows, cols])
                dt = d // SUB
                dr = d % SUB
                for g in range(BG):
                    v = plsc.load_gather(src, [rowbase[g], cols])
                    dst[dt, 0, dr, pl.ds(g * LANES, LANES)] = v + pv

        def fire_writeback(l, b):
            pltpu.async_copy(tbuf[b], out_hbm.at[l, :, pl.ds(wid, 1)], wsem[b])

        def wait_writeback(b):
            pltpu.make_async_copy(
                tbuf[b], out_hbm.at[0, :, pl.ds(wid, 1)], wsem[b]
            ).wait()

        for b in range(NBUF):
            fire_gather(b, b)

        def ring_body(q, carry):
            l0 = NBUF * q
            for b in range(NBUF):
                l = l0 + b
                wait_gather(b)

                @pl.when(q > 0)
                def _():
                    wait_writeback(b)

                transpose_add(l, b)
                fire_writeback(l, b)

                @pl.when(l + NBUF < L)
                def _():
                    fire_gather(l + NBUF, b)

            return carry

        lax.fori_loop(0, L // NBUF, ring_body, 0)
        for b in range(NBUF):
            wait_writeback(b)

    xp = jnp.pad(x.astype(jnp.int32), ((0, 0), (0, LP - L)))
    xt4 = xp.reshape(B // SUB, SUB, LT, LANE).transpose(0, 2, 1, 3)
    out5 = emb_kernel(xt4, token_table, pos_table)
    out = out5.transpose(0, 1, 3, 2, 4).reshape(L, D, B)
    return jnp.transpose(out, (2, 0, 1))


# speedup vs baseline: 1.0145x; 1.0145x over previous
import functools

import jax
import jax.numpy as jnp
from jax import lax
from jax.experimental import pallas as pl
from jax.experimental.pallas import tpu as pltpu
from jax.experimental.pallas import tpu_sc as plsc

NUM_CORES = 2
NUM_SUBCORES = 16
NW = NUM_CORES * NUM_SUBCORES
LANES = 16
SUB = 8
LANE = 128
NBUF = 4


def kernel(x, token_table, pos_table):
    B, L = x.shape
    V, D = token_table.shape
    BW = B // NW
    LP = ((L + LANE - 1) // LANE) * LANE
    LT = LP // LANE
    BT = BW // SUB
    assert B % NW == 0 and BW == LANE and L % NBUF == 0 and D % SUB == 0
    BG = BW // LANES
    DT = D // SUB

    mesh = plsc.VectorSubcoreMesh(core_axis_name="c", subcore_axis_name="s")

    @functools.partial(
        pl.kernel,
        out_type=jax.ShapeDtypeStruct((L, DT, NW, SUB, LANE), jnp.float32),
        mesh=mesh,
        scratch_types=[
            pltpu.VMEM((L, BW), jnp.int32),
            pltpu.VMEM((L, D), jnp.float32),
            [pltpu.VMEM((BW, D), jnp.float32) for _ in range(NBUF)],
            [pltpu.VMEM((DT, 1, SUB, LANE), jnp.float32) for _ in range(NBUF)],
            [pltpu.SemaphoreType.DMA for _ in range(NBUF)],
            [pltpu.SemaphoreType.DMA for _ in range(NBUF)],
        ],
        compiler_params=pltpu.CompilerParams(
            use_tc_tiling_on_sc=False, needs_layout_passes=False
        ),
    )
    def emb_kernel(xt_hbm, tok_hbm, pos_hbm, out_hbm,
                   idx_v, pos_v, inbuf, tbuf, gsem, wsem):
        wid = lax.axis_index("s") * NUM_CORES + lax.axis_index("c")
        iota = lax.iota(jnp.int32, LANES)
        bbase = wid * BW

        pltpu.sync_copy(pos_hbm, pos_v)
        pltpu.sync_copy(xt_hbm.at[:, pl.ds(bbase, BW)], idx_v)

        def fire_gather(l, b):
            pltpu.async_copy(tok_hbm.at[idx_v.at[l]], inbuf[b], gsem[b])

        def wait_gather(b):
            pltpu.make_async_copy(tok_hbm.at[pl.ds(0, BW)], inbuf[b], gsem[b]).wait()

        def transpose_add(l, b):
            src = inbuf[b]
            dst = tbuf[b]
            rowbase = [iota + jnp.int32(g * LANES) for g in range(BG)]
            lrows = jnp.full((LANES,), l, jnp.int32)

            @plsc.parallel_loop(0, D, unroll=2)
            def d_body(d):
                cols = jnp.full((LANES,), d, jnp.int32)
                pv = plsc.load_gather(pos_v, [lrows, cols])
                dt = d // SUB
                dr = d % SUB
                for g in range(BG):
                    v = plsc.load_gather(src, [rowbase[g], cols])
                    dst[dt, 0, dr, pl.ds(g * LANES, LANES)] = v + pv

        def fire_writeback(l, b):
            pltpu.async_copy(tbuf[b], out_hbm.at[l, :, pl.ds(wid, 1)], wsem[b])

        def wait_writeback(b):
            pltpu.make_async_copy(
                tbuf[b], out_hbm.at[0, :, pl.ds(wid, 1)], wsem[b]
            ).wait()

        for b in range(NBUF):
            fire_gather(b, b)

        def ring_body(q, carry):
            l0 = NBUF * q
            for b in range(NBUF):
                l = l0 + b
                wait_gather(b)

                @pl.when(q > 0)
                def _():
                    wait_writeback(b)

                transpose_add(l, b)
                fire_writeback(l, b)

                @pl.when(l + NBUF < L)
                def _():
                    fire_gather(l + NBUF, b)

            return carry

        lax.fori_loop(0, L // NBUF, ring_body, 0)
        for b in range(NBUF):
            wait_writeback(b)

    out5 = emb_kernel(x.T.astype(jnp.int32), token_table, pos_table)
    out = out5.transpose(0, 1, 3, 2, 4).reshape(L, D, B)
    return jnp.transpose(out, (2, 0, 1))
